# baseline (speedup 1.0000x reference)
import functools

import jax
import jax.numpy as jnp
from jax import lax
from jax.experimental import pallas as pl
from jax.experimental.pallas import tpu as pltpu
from jax.experimental.pallas import tpu_sc as plsc

_NUM_COLORS = 256
_B, _H, _W = 16, 512, 512

_NC = 2
_NS = 16
_NW = _NC * _NS
_LANES = 16

_ROWS_PER_W = _H // 2
_CROWS = 16
_CHUNK = _CROWS * _W
_NCHUNK = _ROWS_PER_W // _CROWS
_NVEC = _CHUNK // _LANES
_VPR = _W // _LANES
_UNROLL = 8


def _make_sc_call():
    mesh = plsc.VectorSubcoreMesh(core_axis_name="c", subcore_axis_name="s")

    chunk_i32 = pltpu.VMEM((_CROWS, _W), jnp.int32)
    chunk_f32 = pltpu.VMEM((_CROWS, _W), jnp.float32)

    @functools.partial(
        pl.kernel,
        mesh=mesh,
        out_type=jax.ShapeDtypeStruct((_B, 3, _H, _W), jnp.float32),
        scratch_types=[
            pltpu.VMEM((3, _NUM_COLORS), jnp.float32),
            chunk_i32, chunk_i32,
            chunk_f32, chunk_f32, chunk_f32,
            chunk_f32, chunk_f32, chunk_f32,
            pltpu.SemaphoreType.DMA,
            pltpu.SemaphoreType.DMA,
            pltpu.SemaphoreType.DMA,
        ],
        compiler_params=pltpu.CompilerParams(needs_layout_passes=False),
    )
    def sc_kernel(x_hbm, tbl_hbm, out_hbm, tbl, inb0, inb1,
                  ob00, ob01, ob02, ob10, ob11, ob12,
                  insem, osem0, osem1):
        inbufs = (inb0, inb1)
        outbufs = ((ob00, ob01, ob02), (ob10, ob11, ob12))
        osems = (osem0, osem1)
        wid = lax.axis_index("s") * _NC + lax.axis_index("c")
        img = wid // 2
        half = wid % 2
        row_base = half * _ROWS_PER_W

        pltpu.sync_copy(tbl_hbm, tbl)

        def copy_in(k, slot):
            return pltpu.async_copy(
                x_hbm.at[img, 0, pl.ds(row_base + k * _CROWS, _CROWS), :],
                inbufs[slot],
                insem,
            )

        def copy_out(k, slot):
            handles = []
            for c in range(3):
                handles.append(
                    pltpu.async_copy(
                        outbufs[slot][c],
                        out_hbm.at[img, c, pl.ds(row_base + k * _CROWS, _CROWS), :],
                        osems[slot],
                    )
                )
            return handles

        def compute(slot):
            inb = inbufs[slot]
            obs = outbufs[slot]

            @plsc.parallel_loop(0, _NVEC, 1, unroll=_UNROLL)
            def _body(i):
                r = i // _VPR
                col = (i % _VPR) * _LANES
                v = inb[r, pl.ds(col, _LANES)]
                v = jnp.minimum(jnp.maximum(v, 0), _NUM_COLORS - 1)
                for c in range(3):
                    cvec = jnp.full((_LANES,), c, jnp.int32)
                    obs[c][r, pl.ds(col, _LANES)] = plsc.load_gather(
                        tbl, [cvec, v]
                    )

        in_handles = [None, None]
        out_handles = [None, None]
        in_handles[0] = copy_in(0, 0)
        for k in range(_NCHUNK):
            slot = k % 2
            nxt = (k + 1) % 2
            if k + 1 < _NCHUNK:
                in_handles[nxt] = copy_in(k + 1, nxt)
            in_handles[slot].wait()
            if out_handles[slot] is not None:
                for h in out_handles[slot]:
                    h.wait()
            compute(slot)
            out_handles[slot] = copy_out(k, slot)
        for slot in range(2):
            for h in out_handles[slot]:
                h.wait()

    return sc_kernel


_SC_CALL = _make_sc_call()


@jax.jit
def kernel(input_tensor, colors):
    return _SC_CALL(input_tensor, colors)


# speedup vs baseline: 1.0304x; 1.0304x over previous
import functools

import jax
import jax.numpy as jnp
from jax import lax
from jax.experimental import pallas as pl
from jax.experimental.pallas import tpu as pltpu
from jax.experimental.pallas import tpu_sc as plsc

_NUM_COLORS = 256
_B, _H, _W = 16, 512, 512

_NC = 2
_NS = 16
_NW = _NC * _NS
_LANES = 16

_ROWS_PER_W = _H // 2
_CROWS = 16
_CHUNK = _CROWS * _W
_NCHUNK = _ROWS_PER_W // _CROWS
_NVEC = _CHUNK // _LANES
_VPR = _W // _LANES
_UNROLL = 8


def _make_sc_call():
    mesh = plsc.VectorSubcoreMesh(core_axis_name="c", subcore_axis_name="s")

    chunk_i32 = pltpu.VMEM((_CROWS, _W), jnp.int32)
    chunk_f32 = pltpu.VMEM((_CROWS, _W), jnp.float32)

    @functools.partial(
        pl.kernel,
        mesh=mesh,
        out_type=jax.ShapeDtypeStruct((_B, 3, _H, _W), jnp.float32),
        scratch_types=[
            pltpu.VMEM((3, _NUM_COLORS), jnp.float32),
            pltpu.VMEM((3 * _NUM_COLORS,), jnp.float32),
            chunk_i32, chunk_i32,
            chunk_f32, chunk_f32, chunk_f32,
            chunk_f32, chunk_f32, chunk_f32,
            pltpu.SemaphoreType.DMA,
            pltpu.SemaphoreType.DMA,
            pltpu.SemaphoreType.DMA,
        ],
        compiler_params=pltpu.CompilerParams(needs_layout_passes=False),
    )
    def sc_kernel(x_hbm, tbl_hbm, out_hbm, tbl2d, tbl, inb0, inb1,
                  ob00, ob01, ob02, ob10, ob11, ob12,
                  insem, osem0, osem1):
        inbufs = (inb0, inb1)
        outbufs = ((ob00, ob01, ob02), (ob10, ob11, ob12))
        osems = (osem0, osem1)
        wid = lax.axis_index("s") * _NC + lax.axis_index("c")
        img = wid // 2
        half = wid % 2
        row_base = half * _ROWS_PER_W

        pltpu.sync_copy(tbl_hbm, tbl2d)
        for c in range(3):
            for j in range(_NUM_COLORS // _LANES):
                tbl[pl.ds(c * _NUM_COLORS + j * _LANES, _LANES)] = (
                    tbl2d[c, pl.ds(j * _LANES, _LANES)]
                )

        def copy_in(k, slot):
            return pltpu.async_copy(
                x_hbm.at[img, 0, pl.ds(row_base + k * _CROWS, _CROWS), :],
                inbufs[slot],
                insem,
            )

        def copy_out(k, slot):
            handles = []
            for c in range(3):
                handles.append(
                    pltpu.async_copy(
                        outbufs[slot][c],
                        out_hbm.at[img, c, pl.ds(row_base + k * _CROWS, _CROWS), :],
                        osems[slot],
                    )
                )
            return handles

        def compute(slot):
            inb = inbufs[slot]
            obs = outbufs[slot]

            @plsc.parallel_loop(0, _NVEC, 1, unroll=_UNROLL)
            def _body(i):
                r = i // _VPR
                col = (i % _VPR) * _LANES
                v = inb[r, pl.ds(col, _LANES)]
                v = jnp.minimum(jnp.maximum(v, 0), _NUM_COLORS - 1)
                for c in range(3):
                    obs[c][r, pl.ds(col, _LANES)] = plsc.load_gather(
                        tbl, [v + (c * _NUM_COLORS)]
                    )

        in_handles = [None, None]
        out_handles = [None, None]
        in_handles[0] = copy_in(0, 0)
        for k in range(_NCHUNK):
            slot = k % 2
            nxt = (k + 1) % 2
            if k + 1 < _NCHUNK:
                in_handles[nxt] = copy_in(k + 1, nxt)
            in_handles[slot].wait()
            if out_handles[slot] is not None:
                for h in out_handles[slot]:
                    h.wait()
            compute(slot)
            out_handles[slot] = copy_out(k, slot)
        for slot in range(2):
            for h in out_handles[slot]:
                h.wait()

    return sc_kernel


_SC_CALL = _make_sc_call()


@jax.jit
def kernel(input_tensor, colors):
    return _SC_CALL(input_tensor, colors)


# speedup vs baseline: 1.0585x; 1.0273x over previous
import functools

import jax
import jax.numpy as jnp
from jax import lax
from jax.experimental import pallas as pl
from jax.experimental.pallas import tpu as pltpu
from jax.experimental.pallas import tpu_sc as plsc

_NUM_COLORS = 256
_B, _H, _W = 16, 512, 512

_NC = 2
_NS = 16
_NW = _NC * _NS
_LANES = 16

_ROWS_PER_W = _H // 2
_CROWS = 16
_CHUNK = _CROWS * _W
_NCHUNK = _ROWS_PER_W // _CROWS
_NVEC = _CHUNK // _LANES
_VPR = _W // _LANES
_UNROLL = 8


def _make_sc_call():
    mesh = plsc.VectorSubcoreMesh(core_axis_name="c", subcore_axis_name="s")

    chunk_i32 = pltpu.VMEM((_CROWS, _W), jnp.int32)
    chunk3_f32 = pltpu.VMEM((3, _CROWS, _W), jnp.float32)

    @functools.partial(
        pl.kernel,
        mesh=mesh,
        out_type=jax.ShapeDtypeStruct((_B, 3, _H, _W), jnp.float32),
        scratch_types=[
            pltpu.VMEM((3, _NUM_COLORS), jnp.float32),
            pltpu.VMEM((3 * _NUM_COLORS,), jnp.float32),
            chunk_i32, chunk_i32,
            chunk3_f32, chunk3_f32,
            pltpu.SemaphoreType.DMA,
            pltpu.SemaphoreType.DMA,
            pltpu.SemaphoreType.DMA,
        ],
        compiler_params=pltpu.CompilerParams(needs_layout_passes=False),
    )
    def sc_kernel(x_hbm, tbl_hbm, out_hbm, tbl2d, tbl, inb0, inb1,
                  ob0, ob1, insem, osem0, osem1):
        inbufs = (inb0, inb1)
        outbufs = (ob0, ob1)
        osems = (osem0, osem1)
        wid = lax.axis_index("s") * _NC + lax.axis_index("c")
        img = wid // 2
        half = wid % 2
        row_base = half * _ROWS_PER_W

        def copy_in(k, slot):
            return pltpu.async_copy(
                x_hbm.at[img, 0, pl.ds(row_base + k * _CROWS, _CROWS), :],
                inbufs[slot],
                insem,
            )

        def copy_out(k, slot):
            return [
                pltpu.async_copy(
                    outbufs[slot],
                    out_hbm.at[img, :, pl.ds(row_base + k * _CROWS, _CROWS), :],
                    osems[slot],
                )
            ]

        def compute(slot):
            inb = inbufs[slot]
            outb = outbufs[slot]

            @plsc.parallel_loop(0, _NVEC, 1, unroll=_UNROLL)
            def _body(i):
                r = i // _VPR
                col = (i % _VPR) * _LANES
                v = inb[r, pl.ds(col, _LANES)]
                v = jnp.minimum(jnp.maximum(v, 0), _NUM_COLORS - 1)
                for c in range(3):
                    outb[c, r, pl.ds(col, _LANES)] = plsc.load_gather(
                        tbl, [v + (c * _NUM_COLORS)]
                    )

        in_handles = [None, None]
        out_handles = [None, None]
        in_handles[0] = copy_in(0, 0)
        pltpu.sync_copy(tbl_hbm, tbl2d)
        for c in range(3):
            for j in range(_NUM_COLORS // _LANES):
                tbl[pl.ds(c * _NUM_COLORS + j * _LANES, _LANES)] = (
                    tbl2d[c, pl.ds(j * _LANES, _LANES)]
                )
        for k in range(_NCHUNK):
            slot = k % 2
            nxt = (k + 1) % 2
            if k + 1 < _NCHUNK:
                in_handles[nxt] = copy_in(k + 1, nxt)
            in_handles[slot].wait()
            if out_handles[slot] is not None:
                for h in out_handles[slot]:
                    h.wait()
            compute(slot)
            out_handles[slot] = copy_out(k, slot)
        for slot in range(2):
            for h in out_handles[slot]:
                h.wait()

    return sc_kernel


_SC_CALL = _make_sc_call()


@jax.jit
def kernel(input_tensor, colors):
    return _SC_CALL(input_tensor, colors)
